# Initial kernel scaffold; baseline (speedup 1.0000x reference)
import jax
import jax.numpy as jnp
from jax.experimental import pallas as pl


def kernel(x, edge_index, Wc, bc, W1, b1, W2, b2, W3, b3):
    raise NotImplementedError("write your pallas kernel here")


# speedup vs baseline: 14.6326x; 14.6326x over previous
import functools

import jax
import jax.numpy as jnp
from jax import lax
from jax.experimental import pallas as pl
from jax.experimental.pallas import tpu as pltpu
from jax.experimental.pallas import tpu_sc as plsc

NC = 2
NS = 16
NW = NC * NS
L = 16
CHUNK = 128


def _sc_mesh():
    return plsc.VectorSubcoreMesh(core_axis_name="c", subcore_axis_name="s")


def _degree_call(src3, dst3, nacc):
    nchunk = src3.shape[1]
    rows = nacc // NS

    @functools.partial(
        pl.kernel,
        out_type=jax.ShapeDtypeStruct((2, NC, nacc), jnp.float32),
        mesh=_sc_mesh(),
        scratch_types=[
            pltpu.VMEM((nchunk, CHUNK), jnp.int32),
            pltpu.VMEM((nchunk, CHUNK), jnp.int32),
            pltpu.VMEM((CHUNK,), jnp.float32),
            pltpu.VMEM((rows,), jnp.float32),
            pltpu.VMEM_SHARED((nacc,), jnp.float32),
            pltpu.VMEM_SHARED((nacc,), jnp.float32),
        ],
    )
    def deg_kernel(src_hbm, dst_hbm, degs_out, idx_s, idx_d, ones_v, zbuf,
                   dsrc_sh, ddst_sh):
        cid = lax.axis_index("c")
        sid = lax.axis_index("s")
        wid = cid * NS + sid

        def fill_ones(i, _):
            ones_v[pl.ds(i * L, L)] = jnp.ones((L,), jnp.float32)
            return 0

        lax.fori_loop(0, CHUNK // L, fill_ones, 0)

        def fill_zero(i, _):
            zbuf[pl.ds(i * L, L)] = jnp.zeros((L,), jnp.float32)
            return 0

        lax.fori_loop(0, rows // L, fill_zero, 0)

        pltpu.sync_copy(zbuf, dsrc_sh.at[pl.ds(sid * rows, rows)])
        pltpu.sync_copy(zbuf, ddst_sh.at[pl.ds(sid * rows, rows)])
        plsc.subcore_barrier()

        pltpu.sync_copy(src_hbm.at[wid], idx_s)
        pltpu.sync_copy(dst_hbm.at[wid], idx_d)

        def body(j, _):
            pltpu.sync_copy(ones_v, dsrc_sh.at[idx_s.at[j]], add=True)
            pltpu.sync_copy(ones_v, ddst_sh.at[idx_d.at[j]], add=True)
            return 0

        lax.fori_loop(0, nchunk, body, 0)
        plsc.subcore_barrier()

        pltpu.sync_copy(dsrc_sh.at[pl.ds(sid * rows, rows)],
                        degs_out.at[0, cid, pl.ds(sid * rows, rows)])
        pltpu.sync_copy(ddst_sh.at[pl.ds(sid * rows, rows)],
                        degs_out.at[1, cid, pl.ds(sid * rows, rows)])

    return deg_kernel(src3, dst3)


def _scatter_call(h2, src3, dst3, nacc):
    nchunk = src3.shape[1]
    d = h2.shape[1]
    rows = nacc // NS

    @functools.partial(
        pl.kernel,
        out_type=jax.ShapeDtypeStruct((NC, nacc, d), jnp.float32),
        mesh=_sc_mesh(),
        scratch_types=[
            pltpu.VMEM((nchunk, CHUNK), jnp.int32),
            pltpu.VMEM((nchunk, CHUNK), jnp.int32),
            pltpu.VMEM((CHUNK, d), jnp.float32),
            pltpu.VMEM_SHARED((nacc, d), jnp.float32),
            pltpu.SemaphoreType.DMA,
        ],
    )
    def scat_kernel(h2_hbm, src_hbm, dst_hbm, part_out, idx_s, idx_d,
                    rows_v, acc_sh, sem):
        cid = lax.axis_index("c")
        sid = lax.axis_index("s")
        wid = cid * NS + sid

        def fill_zero(i, _):
            rows_v[i // (d // L), pl.ds((i % (d // L)) * L, L)] = (
                jnp.zeros((L,), jnp.float32))
            return 0

        lax.fori_loop(0, CHUNK * (d // L), fill_zero, 0)

        for k in range(rows // CHUNK):
            pltpu.sync_copy(rows_v, acc_sh.at[pl.ds(sid * rows + k * CHUNK, CHUNK)])
        plsc.subcore_barrier()

        pltpu.sync_copy(src_hbm.at[wid], idx_s)
        pltpu.sync_copy(dst_hbm.at[wid], idx_d)

        def body(j, _):
            pltpu.async_copy(h2_hbm.at[idx_s.at[j]], rows_v, sem).wait()
            pltpu.sync_copy(rows_v, acc_sh.at[idx_d.at[j]], add=True)
            return 0

        lax.fori_loop(0, nchunk, body, 0)
        plsc.subcore_barrier()

        pltpu.sync_copy(acc_sh.at[pl.ds(sid * rows, rows)],
                        part_out.at[cid, pl.ds(sid * rows, rows)])

    return scat_kernel(h2, src3, dst3)


def _h2_call(x, wc, deg_src3):
    n, d = x.shape
    h1 = wc.shape[1]
    blk = 2000

    def body(x_ref, wc_ref, deg_ref, out_ref):
        h = jnp.dot(x_ref[...], wc_ref[...], preferred_element_type=jnp.float32)
        deg = deg_ref[0] + deg_ref[1] + 1.0
        out_ref[...] = h * lax.rsqrt(deg)

    return pl.pallas_call(
        body,
        grid=(n // blk,),
        in_specs=[
            pl.BlockSpec((blk, d), lambda b: (b, 0)),
            pl.BlockSpec((d, h1), lambda b: (0, 0)),
            pl.BlockSpec((2, blk, 1), lambda b: (0, b, 0)),
        ],
        out_specs=pl.BlockSpec((blk, h1), lambda b: (b, 0)),
        out_shape=jax.ShapeDtypeStruct((n, h1), jnp.float32),
    )(x, wc, deg_src3)


def _final_call(part, h2, deg_dst3, bc2, w1, b1_2, w2, b2_2, w3, b3_2):
    n, h1 = h2.shape
    f1 = w1.shape[1]
    f2 = w2.shape[1]
    out = w3.shape[1]
    blk = 2000
    nblk = n // blk

    def body(p_ref, h2_ref, deg_ref, bc_ref, w1_ref, b1_ref, w2_ref, b2_ref,
             w3_ref, b3_ref, out_ref, mx_ref):
        b = pl.program_id(0)
        tot = p_ref[0] + p_ref[1] + h2_ref[...]
        deg = deg_ref[0] + deg_ref[1] + 1.0
        agg = tot * lax.rsqrt(deg) + bc_ref[...]
        r = jnp.max(jnp.maximum(agg, 0.0), axis=0, keepdims=True)

        @pl.when(b == 0)
        def _():
            mx_ref[...] = r

        @pl.when(b > 0)
        def _():
            mx_ref[...] = jnp.maximum(mx_ref[...], r)

        @pl.when(b == nblk - 1)
        def _():
            hg = mx_ref[...]
            a = jnp.dot(hg, w1_ref[...], preferred_element_type=jnp.float32)
            a = jnp.maximum(a + b1_ref[...], 0.0)
            a = jnp.dot(a, w2_ref[...], preferred_element_type=jnp.float32)
            a = jnp.maximum(a + b2_ref[...], 0.0)
            a = jnp.dot(a, w3_ref[...], preferred_element_type=jnp.float32)
            out_ref[...] = a + b3_ref[...]

    return pl.pallas_call(
        body,
        grid=(nblk,),
        in_specs=[
            pl.BlockSpec((NC, blk, h1), lambda b: (0, b, 0)),
            pl.BlockSpec((blk, h1), lambda b: (b, 0)),
            pl.BlockSpec((2, blk, 1), lambda b: (0, b, 0)),
            pl.BlockSpec((1, h1), lambda b: (0, 0)),
            pl.BlockSpec((h1, f1), lambda b: (0, 0)),
            pl.BlockSpec((1, f1), lambda b: (0, 0)),
            pl.BlockSpec((f1, f2), lambda b: (0, 0)),
            pl.BlockSpec((1, f2), lambda b: (0, 0)),
            pl.BlockSpec((f2, out), lambda b: (0, 0)),
            pl.BlockSpec((1, out), lambda b: (0, 0)),
        ],
        out_specs=pl.BlockSpec((1, out), lambda b: (0, 0)),
        out_shape=jax.ShapeDtypeStruct((1, out), jnp.float32),
        scratch_shapes=[pltpu.VMEM((1, h1), jnp.float32)],
    )(part, h2, deg_dst3, bc2, w1, b1_2, w2, b2_2, w3, b3_2)


def kernel(x, edge_index, Wc, bc, W1, b1, W2, b2, W3, b3):
    n = x.shape[0]
    e = edge_index.shape[1]

    nacc = ((n + 1 + NS * CHUNK - 1) // (NS * CHUNK)) * (NS * CHUNK)
    step = NW * CHUNK
    epad = ((e + step - 1) // step) * step
    nchunk = epad // (NW * CHUNK)
    padn = epad - e

    src = edge_index[0]
    dst = edge_index[1]
    pad_ignore = jnp.full((padn,), n, dtype=jnp.int32)
    src_deg3 = jnp.concatenate([src, pad_ignore]).reshape(NW, nchunk, CHUNK)
    src_gat3 = jnp.concatenate(
        [src, jnp.zeros((padn,), dtype=jnp.int32)]).reshape(NW, nchunk, CHUNK)
    dst3 = jnp.concatenate([dst, pad_ignore]).reshape(NW, nchunk, CHUNK)

    degs = _degree_call(src_deg3, dst3, nacc)
    deg_src3 = degs[0][:, :n].reshape(NC, n, 1)
    deg_dst3 = degs[1][:, :n].reshape(NC, n, 1)

    h2 = _h2_call(x, Wc, deg_src3)
    part = _scatter_call(h2, src_gat3, dst3, nacc)

    res = _final_call(
        part, h2, deg_dst3,
        bc.reshape(1, -1), W1, b1.reshape(1, -1),
        W2, b2.reshape(1, -1), W3, b3.reshape(1, -1))
    return res[0]
